# baseline (speedup 1.0000x reference)
import jax
import jax.numpy as jnp
from jax import lax
from jax.experimental import pallas as pl
from jax.experimental.pallas import tpu as pltpu
from jax.experimental.pallas import tpu_sc as plsc

B, C, N_BIN = 256, 128, 360
MAX_SHIFT = 15
LOWER_BIN = 15
N_OUT = N_BIN - 2 * MAX_SHIFT

NUM_WORKERS = 32
B_PER_W = B // NUM_WORKERS
J_CHUNK = 16
N_G = 21
NBUF = 4
LAG = 2


def _chunk_j0(g):
    return N_OUT - J_CHUNK if g == N_G - 1 else J_CHUNK * g


def _sc_body(rows_hbm, ns_hbm, x_hbm, xs_hbm,
             ns2, idx0, idx1, idx2, idx3,
             odx0, odx1, odx2, odx3, gb0, gb1, gb2, gb3,
             gsem0, gsem1, gsem2, gsem3,
             osem0, osem1, osem2, osem3):
    wid = lax.axis_index("s") * 2 + lax.axis_index("c")
    base = wid * B_PER_W

    pltpu.sync_copy(ns_hbm.at[pl.ds(base, B_PER_W)], ns2.at[pl.ds(0, B_PER_W)])
    pltpu.sync_copy(ns_hbm.at[pl.ds(base, B_PER_W)], ns2.at[pl.ds(B_PER_W, B_PER_W)])
    ns_vec = ns2[...]

    lanes = lax.iota(jnp.int32, 16)
    jv = lanes >> 3
    dbv = lanes & 7
    rowbase = (base + dbv) * N_BIN + jv
    base_x = rowbase + LOWER_BIN
    base_s = rowbase + (LOWER_BIN - ns_vec)
    obase = (base + dbv) + B * jv

    idxb = (idx0, idx1, idx2, idx3)
    odxb = (odx0, odx1, odx2, odx3)
    gb = (gb0, gb1, gb2, gb3)
    gsem = (gsem0, gsem1, gsem2, gsem3)
    osem = (osem0, osem1, osem2, osem3)
    outs = (x_hbm, xs_hbm)
    bases = (base_x, base_s)

    tasks = [(win, g) for win in range(2) for g in range(N_G)]
    T = len(tasks)

    def out_copy(win, slot):
        return pltpu.make_async_copy(
            gb[slot], outs[win].at[odxb[slot]], osem[slot]
        )

    for t in range(T + LAG):
        slot = t % NBUF
        if t < T:
            win, g = tasks[t]
            j0 = _chunk_j0(g)
            if t >= NBUF:
                pwin, pg = tasks[t - NBUF]
                out_copy(pwin, slot).wait()
            bvec = bases[win] + j0
            for m in range(8):
                idxb[slot][pl.ds(16 * m, 16)] = bvec + 2 * m
            pltpu.async_copy(rows_hbm.at[idxb[slot]], gb[slot], gsem[slot])
        if t >= LAG:
            tt = t - LAG
            slot2 = tt % NBUF
            win2, g2 = tasks[tt]
            jj0 = _chunk_j0(g2)
            pltpu.make_async_copy(rows_hbm.at[idxb[slot2]], gb[slot2], gsem[slot2]).wait()
            ovec = obase + B * jj0
            for m in range(8):
                odxb[slot2][pl.ds(16 * m, 16)] = ovec + 2 * B * m
            out_copy(win2, slot2).start()

    for tt in range(max(0, T - NBUF), T):
        slot = tt % NBUF
        win, g = tasks[tt]
        out_copy(win, slot).wait()


def kernel(spec, n_shifts):
    ns32 = n_shifts.astype(jnp.int32)
    rows = jnp.transpose(spec, (0, 2, 1)).reshape(B * N_BIN, C)
    mesh = plsc.VectorSubcoreMesh(core_axis_name="c", subcore_axis_name="s")
    x_t, xs_t = pl.kernel(
        _sc_body,
        out_type=(
            jax.ShapeDtypeStruct((N_OUT * B, C), jnp.float32),
            jax.ShapeDtypeStruct((N_OUT * B, C), jnp.float32),
        ),
        mesh=mesh,
        compiler_params=pltpu.CompilerParams(
            use_tc_tiling_on_sc=True, needs_layout_passes=False
        ),
        scratch_types=(
            pltpu.VMEM((16,), jnp.int32),
            pltpu.VMEM((J_CHUNK * B_PER_W,), jnp.int32),
            pltpu.VMEM((J_CHUNK * B_PER_W,), jnp.int32),
            pltpu.VMEM((J_CHUNK * B_PER_W,), jnp.int32),
            pltpu.VMEM((J_CHUNK * B_PER_W,), jnp.int32),
            pltpu.VMEM((J_CHUNK * B_PER_W,), jnp.int32),
            pltpu.VMEM((J_CHUNK * B_PER_W,), jnp.int32),
            pltpu.VMEM((J_CHUNK * B_PER_W,), jnp.int32),
            pltpu.VMEM((J_CHUNK * B_PER_W,), jnp.int32),
            pltpu.VMEM((J_CHUNK * B_PER_W, C), jnp.float32),
            pltpu.VMEM((J_CHUNK * B_PER_W, C), jnp.float32),
            pltpu.VMEM((J_CHUNK * B_PER_W, C), jnp.float32),
            pltpu.VMEM((J_CHUNK * B_PER_W, C), jnp.float32),
            pltpu.SemaphoreType.DMA,
            pltpu.SemaphoreType.DMA,
            pltpu.SemaphoreType.DMA,
            pltpu.SemaphoreType.DMA,
            pltpu.SemaphoreType.DMA,
            pltpu.SemaphoreType.DMA,
            pltpu.SemaphoreType.DMA,
            pltpu.SemaphoreType.DMA,
        ),
    )(rows, ns32)
    x = jnp.transpose(x_t.reshape(N_OUT, B, C), (1, 2, 0))
    xs = jnp.transpose(xs_t.reshape(N_OUT, B, C), (1, 2, 0))
    return (x, xs, n_shifts)


# speedup vs baseline: 1.0299x; 1.0299x over previous
import jax
import jax.numpy as jnp
from jax import lax
from jax.experimental import pallas as pl
from jax.experimental.pallas import tpu as pltpu
from jax.experimental.pallas import tpu_sc as plsc

B, C, N_BIN = 256, 128, 360
MAX_SHIFT = 15
LOWER_BIN = 15
N_OUT = N_BIN - 2 * MAX_SHIFT

NUM_WORKERS = 32
B_PER_W = B // NUM_WORKERS
J_CHUNK = 16
N_G = 21
NBUF = 6
LAG = 3


def _chunk_j0(g):
    return N_OUT - J_CHUNK if g == N_G - 1 else J_CHUNK * g


def _sc_body(rows_hbm, ns_hbm, x_hbm, xs_hbm,
             ns2, idx0, idx1, idx2, idx3, idx4, idx5, gb0, gb1, gb2, gb3, gb4, gb5,
             gsem0, gsem1, gsem2, gsem3, gsem4, gsem5,
             osem0, osem1, osem2, osem3, osem4, osem5):
    wid = lax.axis_index("s") * 2 + lax.axis_index("c")
    base = wid * B_PER_W

    pltpu.sync_copy(ns_hbm.at[pl.ds(base, B_PER_W)], ns2.at[pl.ds(0, B_PER_W)])
    pltpu.sync_copy(ns_hbm.at[pl.ds(base, B_PER_W)], ns2.at[pl.ds(B_PER_W, B_PER_W)])
    ns_vec = ns2[...]

    lanes = lax.iota(jnp.int32, 16)
    jv = lanes >> 3
    dbv = lanes & 7
    rowbase = (base + dbv) * N_BIN + jv
    base_x = rowbase + LOWER_BIN
    base_s = rowbase + (LOWER_BIN - ns_vec)

    idxb = (idx0, idx1, idx2, idx3, idx4, idx5)
    gb = (gb0, gb1, gb2, gb3, gb4, gb5)
    gsem = (gsem0, gsem1, gsem2, gsem3, gsem4, gsem5)
    osem = (osem0, osem1, osem2, osem3, osem4, osem5)
    outs = (x_hbm, xs_hbm)
    bases = (base_x, base_s)

    tasks = [(win, g) for win in range(2) for g in range(N_G)]
    T = len(tasks)

    def out_copy(win, j, slot, m):
        return pltpu.make_async_copy(
            gb[slot].at[pl.ds(8 * m, 8), :],
            outs[win].at[j, pl.ds(base, B_PER_W), :],
            osem[slot],
        )

    for t in range(T + LAG):
        slot = t % NBUF
        if t < T:
            win, g = tasks[t]
            j0 = _chunk_j0(g)
            if t >= NBUF:
                pwin, pg = tasks[t - NBUF]
                pj0 = _chunk_j0(pg)
                for m in range(J_CHUNK):
                    out_copy(pwin, pj0 + m, slot, m).wait()
            bvec = bases[win] + j0
            for m in range(8):
                idxb[slot][pl.ds(16 * m, 16)] = bvec + 2 * m
            pltpu.async_copy(rows_hbm.at[idxb[slot]], gb[slot], gsem[slot])
        if t >= LAG:
            tt = t - LAG
            slot2 = tt % NBUF
            win2, g2 = tasks[tt]
            jj0 = _chunk_j0(g2)
            pltpu.make_async_copy(rows_hbm.at[idxb[slot2]], gb[slot2], gsem[slot2]).wait()
            for m in range(J_CHUNK):
                out_copy(win2, jj0 + m, slot2, m).start()

    for tt in range(max(0, T - NBUF), T):
        slot = tt % NBUF
        win, g = tasks[tt]
        j0 = _chunk_j0(g)
        for m in range(J_CHUNK):
            out_copy(win, j0 + m, slot, m).wait()


def kernel(spec, n_shifts):
    ns32 = n_shifts.astype(jnp.int32)
    rows = jnp.transpose(spec, (0, 2, 1)).reshape(B * N_BIN, C)
    mesh = plsc.VectorSubcoreMesh(core_axis_name="c", subcore_axis_name="s")
    x_t, xs_t = pl.kernel(
        _sc_body,
        out_type=(
            jax.ShapeDtypeStruct((N_OUT, B, C), jnp.float32),
            jax.ShapeDtypeStruct((N_OUT, B, C), jnp.float32),
        ),
        mesh=mesh,
        compiler_params=pltpu.CompilerParams(
            use_tc_tiling_on_sc=True, needs_layout_passes=False
        ),
        scratch_types=(
            pltpu.VMEM((16,), jnp.int32),
            pltpu.VMEM((J_CHUNK * B_PER_W,), jnp.int32),
            pltpu.VMEM((J_CHUNK * B_PER_W,), jnp.int32),
            pltpu.VMEM((J_CHUNK * B_PER_W,), jnp.int32),
            pltpu.VMEM((J_CHUNK * B_PER_W,), jnp.int32),
            pltpu.VMEM((J_CHUNK * B_PER_W,), jnp.int32),
            pltpu.VMEM((J_CHUNK * B_PER_W,), jnp.int32),
            pltpu.VMEM((J_CHUNK * B_PER_W, C), jnp.float32),
            pltpu.VMEM((J_CHUNK * B_PER_W, C), jnp.float32),
            pltpu.VMEM((J_CHUNK * B_PER_W, C), jnp.float32),
            pltpu.VMEM((J_CHUNK * B_PER_W, C), jnp.float32),
            pltpu.VMEM((J_CHUNK * B_PER_W, C), jnp.float32),
            pltpu.VMEM((J_CHUNK * B_PER_W, C), jnp.float32),
            pltpu.SemaphoreType.DMA,
            pltpu.SemaphoreType.DMA,
            pltpu.SemaphoreType.DMA,
            pltpu.SemaphoreType.DMA,
            pltpu.SemaphoreType.DMA,
            pltpu.SemaphoreType.DMA,
            pltpu.SemaphoreType.DMA,
            pltpu.SemaphoreType.DMA,
            pltpu.SemaphoreType.DMA,
            pltpu.SemaphoreType.DMA,
            pltpu.SemaphoreType.DMA,
            pltpu.SemaphoreType.DMA,
        ),
    )(rows, ns32)
    x = jnp.transpose(x_t, (1, 2, 0))
    xs = jnp.transpose(xs_t, (1, 2, 0))
    return (x, xs, n_shifts)
